# Initial kernel scaffold; baseline (speedup 1.0000x reference)
import jax
import jax.numpy as jnp
from jax.experimental import pallas as pl


def kernel(输入, 标签):
    raise NotImplementedError("write your pallas kernel here")


# speedup vs baseline: 2.0223x; 2.0223x over previous
import functools

import jax
import jax.numpy as jnp
from jax import lax
from jax.experimental import pallas as pl
from jax.experimental.pallas import tpu as pltpu
from jax.experimental.pallas import tpu_sc as plsc

L = 16
NSUB = 16
PER = 3840
NPAD = NSUB * PER
CH = PER // L
SEG = PER + L
NCLS = 300
HI0 = 0x7F800000

_f32 = jnp.float32
_i32 = jnp.int32


def _softplus16(x):
    z = jnp.exp(-jnp.abs(x))
    w = z / (z + _f32(2.0))
    w2 = w * w
    p = _f32(1.0 / 13.0)
    p = _f32(1.0 / 11.0) + w2 * p
    p = _f32(1.0 / 9.0) + w2 * p
    p = _f32(1.0 / 7.0) + w2 * p
    p = _f32(1.0 / 5.0) + w2 * p
    p = _f32(1.0 / 3.0) + w2 * p
    p = _f32(1.0) + w2 * p
    return jnp.maximum(x, _f32(0.0)) + _f32(2.0) * w * p


@functools.cache
def _build():
    mesh = plsc.VectorSubcoreMesh(core_axis_name="c", subcore_axis_name="s")

    @functools.partial(
        pl.kernel,
        out_type=jax.ShapeDtypeStruct((L,), _f32),
        mesh=mesh,
        compiler_params=pltpu.CompilerParams(needs_layout_passes=False),
        scratch_types=[
            pltpu.VMEM((PER,), _f32),
            pltpu.VMEM((PER,), _f32),
            pltpu.VMEM((PER,), _i32),
            pltpu.VMEM((SEG,), _f32),
            pltpu.VMEM((NSUB * SEG,), _f32),
            pltpu.VMEM((NSUB * L,), _f32),
            pltpu.VMEM((NSUB * L,), _i32),
            pltpu.VMEM((NSUB * L,), _i32),
            pltpu.VMEM((L,), _f32),
            pltpu.VMEM((L,), _i32),
            pltpu.VMEM((L,), _i32),
            pltpu.VMEM((L,), _f32),
            pltpu.VMEM_SHARED((NSUB * SEG,), _f32),
            pltpu.VMEM_SHARED((NSUB * L,), _f32),
            pltpu.VMEM_SHARED((NSUB * L,), _i32),
            pltpu.VMEM_SHARED((NSUB * L,), _i32),
        ],
    )
    def k(l0_hbm, l1_hbm, lab_hbm, out_hbm,
          l0_v, l1_v, lab_v, bgbuf, dense, meta_fg_v, meta_nfg_v,
          meta_off_v, stage_fg, stage_nfg, stage_off, outbuf,
          sh_bg, sh_fg, sh_nfg, sh_off):
        cid = lax.axis_index("c")
        sid = lax.axis_index("s")

        @pl.when(cid == 0)
        def _core0():
            zf = jnp.zeros((L,), _f32)
            zi = jnp.zeros((L,), _i32)
            lane = lax.broadcasted_iota(_i32, (L,), 0)

            base = sid * PER
            pltpu.sync_copy(l0_hbm.at[pl.ds(base, PER)], l0_v)
            pltpu.sync_copy(l1_hbm.at[pl.ds(base, PER)], l1_v)
            pltpu.sync_copy(lab_hbm.at[pl.ds(base, PER)], lab_v)

            def body(i, carry):
                off, fg_acc, nfg_acc = carry
                sl = pl.ds(i * L, L)
                x0 = l0_v[sl]
                x1 = l1_v[sl]
                lb = lab_v[sl]
                dd = x1 - x0
                is_fg = lb == 1
                is_bg = lb == 0
                loss = _softplus16(jnp.where(is_fg, -dd, dd))
                fg_acc = fg_acc + jnp.where(is_fg, loss, _f32(0.0))
                nfg_acc = nfg_acc + jnp.where(is_fg, _i32(1), _i32(0))
                bg_i = jnp.where(is_bg, _i32(1), _i32(0))
                pos = off + lax.cumsum(bg_i, axis=0) - _i32(1)
                plsc.store_scatter(bgbuf, [pos], loss, mask=is_bg)
                return off + jnp.sum(bg_i, dtype=_i32), fg_acc, nfg_acc

            off, fg_acc, nfg_acc = lax.fori_loop(
                _i32(0), _i32(CH), body, (_i32(0), zf, zi))
            plsc.store_scatter(bgbuf, [off + lane],
                               jnp.full((L,), -jnp.inf, _f32))

            stage_fg[...] = fg_acc
            stage_nfg[...] = nfg_acc
            stage_off[...] = zi + off
            pltpu.sync_copy(bgbuf, sh_bg.at[pl.ds(sid * SEG, SEG)])
            pltpu.sync_copy(stage_fg, sh_fg.at[pl.ds(sid * L, L)])
            pltpu.sync_copy(stage_nfg, sh_nfg.at[pl.ds(sid * L, L)])
            pltpu.sync_copy(stage_off, sh_off.at[pl.ds(sid * L, L)])
            plsc.subcore_barrier()

            @pl.when(sid == 0)
            def _merge():
                pltpu.sync_copy(sh_fg, meta_fg_v)
                pltpu.sync_copy(sh_nfg, meta_nfg_v)
                pltpu.sync_copy(sh_off, meta_off_v)

                def red(w_, carry):
                    fg_v, nfg_v = carry
                    slw = pl.ds(w_ * L, L)
                    return fg_v + meta_fg_v[slw], nfg_v + meta_nfg_v[slw]

                fg_v, nfg_v = lax.fori_loop(_i32(0), _i32(NSUB), red, (zf, zi))
                fg_sum = jnp.sum(fg_v)
                n_fg = jnp.sum(nfg_v, dtype=_i32)

                def gather_w(w_, carry):
                    g, nbg = carry
                    offw = jnp.max(meta_off_v[pl.ds(w_ * L, L)])
                    nch = lax.shift_right_logical(offw + _i32(L - 1), _i32(4))

                    def cp(j, gg):
                        pltpu.sync_copy(
                            sh_bg.at[pl.ds(w_ * SEG + j * L, L)],
                            dense.at[pl.ds(gg * L, L)])
                        return gg + _i32(1)

                    g = lax.fori_loop(_i32(0), nch, cp, g)
                    return g, nbg + offw

                G, n_bg = lax.fori_loop(_i32(0), _i32(NSUB), gather_w,
                                        (_i32(0), _i32(0)))
                K = _i32(NCLS) - n_fg

                def bs(_, carry):
                    lo, hi = carry
                    mid = lo + lax.shift_right_logical(hi - lo, _i32(1))
                    tv = plsc.bitcast(zi + mid, _f32)

                    def cb(j, acc):
                        v = dense[pl.ds(j * L, L)]
                        return acc + jnp.where(v >= tv, _i32(1), _i32(0))

                    c = jnp.sum(lax.fori_loop(_i32(0), G, cb, zi), dtype=_i32)
                    pred = c >= K
                    return (jnp.where(pred, mid, lo),
                            jnp.where(pred, hi, mid))

                lo, _hi = lax.fori_loop(_i32(0), _i32(31), bs,
                                        (_i32(0), _i32(HI0)))
                tv = plsc.bitcast(zi + lo, _f32)

                def fin(j, carry):
                    cv, sv = carry
                    v = dense[pl.ds(j * L, L)]
                    m = v > tv
                    return (cv + jnp.where(m, _i32(1), _i32(0)),
                            sv + jnp.where(m, v, _f32(0.0)))

                cv, sv = lax.fori_loop(_i32(0), G, fin, (zi, zf))
                c_gt = jnp.sum(cv, dtype=_i32)
                s_gt = jnp.sum(sv)
                t_s = jnp.max(tv)
                bg_main = s_gt + (K - c_gt).astype(_f32) * t_s
                bg_sum = jnp.where(
                    K <= _i32(0), _f32(0.0),
                    jnp.where(K > n_bg, _f32(-jnp.inf), bg_main))
                outbuf[...] = (zf + (fg_sum + bg_sum)) / (zf + _f32(NCLS))
                pltpu.sync_copy(outbuf, out_hbm)

    return k


def kernel(输入, 标签):
    logits = 输入[0]
    labels = 标签[0, 0].astype(_i32)
    n = logits.shape[0]
    pad = NPAD - n
    l0 = jnp.concatenate([logits[:, 0], jnp.zeros((pad,), _f32)])
    l1 = jnp.concatenate([logits[:, 1], jnp.zeros((pad,), _f32)])
    lab = jnp.concatenate([labels, jnp.full((pad,), 2, _i32)])
    out = _build()(l0, l1, lab)
    return out[0]
